# baseline (speedup 1.0000x reference)
import functools

import jax
import jax.numpy as jnp
from jax import lax
from jax.experimental import pallas as pl
from jax.experimental.pallas import tpu as pltpu
from jax.experimental.pallas import tpu_sc as plsc

E = 64
DIN = 768
DOUT = 768
T = 32768
BT = 256
CAP = T + E * BT
G = CAP // BT

NW = 32
CHUNK = 32
NCH = T // NW // CHUNK
NBUF = 4


def _make_row_shuffle(n_table, n_out, d):
    mesh = plsc.VectorSubcoreMesh(core_axis_name="c", subcore_axis_name="s")

    @functools.partial(
        pl.kernel,
        mesh=mesh,
        out_type=jax.ShapeDtypeStruct((n_out, d), jnp.float32),
        scratch_types=[
            pltpu.VMEM((NCH, CHUNK), jnp.int32),
            pltpu.VMEM((NCH, CHUNK), jnp.int32),
            pltpu.VMEM((NBUF, CHUNK, d), jnp.float32),
            pltpu.SemaphoreType.DMA,
            pltpu.SemaphoreType.DMA,
            pltpu.SemaphoreType.DMA,
            pltpu.SemaphoreType.DMA,
            pltpu.SemaphoreType.DMA,
        ],
    )
    def k(table_hbm, src_hbm, dst_hbm, out_hbm, src_v, dst_v, rows_v,
          isem, gsem0, gsem1, ssem0, ssem1):
        gsem = (gsem0, gsem1)
        ssem = (ssem0, ssem1)
        wid = lax.axis_index("s") * 2 + lax.axis_index("c")
        row0 = wid * NCH
        pltpu.async_copy(src_hbm.at[pl.ds(row0, NCH)], src_v, isem)
        pltpu.async_copy(dst_hbm.at[pl.ds(row0, NCH)], dst_v, isem).wait()
        pltpu.make_async_copy(src_hbm.at[pl.ds(row0, NCH)], src_v, isem).wait()

        hg = [None] * NCH
        hs = [None] * NCH
        for c in range(2):
            hg[c] = pltpu.async_copy(
                table_hbm.at[src_v.at[c]], rows_v.at[c], gsem[c % 2])
        for c in range(NCH):
            if c >= 2:
                hs[c - 2].wait()
            hg[c].wait()
            if c + 2 < NCH:
                hg[c + 2] = pltpu.async_copy(
                    table_hbm.at[src_v.at[c + 2]],
                    rows_v.at[(c + 2) % NBUF], gsem[c % 2])
            hs[c] = pltpu.async_copy(
                rows_v.at[c % NBUF], out_hbm.at[dst_v.at[c]], ssem[c % 2])
        hs[NCH - 2].wait()
        hs[NCH - 1].wait()

    return k


_shuffle_in = _make_row_shuffle(T, CAP, DIN)
_shuffle_out = _make_row_shuffle(CAP, T, DOUT)


def _mm_body(bx_ref, be_ref, x_ref, w_ref, y_ref):
    y_ref[...] = lax.dot_general(
        x_ref[...], w_ref[0],
        dimension_numbers=(((1,), (1,)), ((), ())),
        preferred_element_type=jnp.float32,
    )


_grouped_mm = pl.pallas_call(
    _mm_body,
    grid_spec=pltpu.PrefetchScalarGridSpec(
        num_scalar_prefetch=2,
        grid=(G,),
        in_specs=[
            pl.BlockSpec((BT, DIN), lambda g, bx, be: (bx[g], 0)),
            pl.BlockSpec((1, DOUT, DIN), lambda g, bx, be: (be[g], 0, 0)),
        ],
        out_specs=pl.BlockSpec((BT, DOUT), lambda g, bx, be: (bx[g], 0)),
    ),
    out_shape=jax.ShapeDtypeStruct((CAP, DOUT), jnp.float32),
    compiler_params=pltpu.CompilerParams(
        dimension_semantics=("arbitrary",),
    ),
)


def kernel(inp, gate, weight):
    gate = gate.astype(jnp.int32)
    k = jnp.arange(T, dtype=jnp.int32)
    key = jnp.sort((gate << 16) | k)
    s = key & 0xFFFF
    sg = key >> 16
    counts = jnp.bincount(gate, length=E).astype(jnp.int32)
    raw_off = jnp.concatenate([jnp.zeros((1,), jnp.int32),
                               jnp.cumsum(counts)[:-1].astype(jnp.int32)])
    padded = ((counts + BT - 1) // BT) * BT
    pad_off = jnp.concatenate([jnp.zeros((1,), jnp.int32),
                               jnp.cumsum(padded)[:-1].astype(jnp.int32)])
    inc = padded - counts
    bnd = raw_off[1:]
    dest = k + jnp.sum(
        (bnd[None, :] <= k[:, None]) * inc[None, :-1],
        axis=1, dtype=jnp.int32)
    cum_blocks = jnp.cumsum(padded // BT).astype(jnp.int32)
    g_ids = jnp.arange(G, dtype=jnp.int32)
    block_expert = jnp.sum(
        cum_blocks[None, :] <= g_ids[:, None], axis=1, dtype=jnp.int32)
    block_expert = jnp.minimum(block_expert, E - 1)
    valid = g_ids < cum_blocks[-1]
    bxa = jnp.where(valid, g_ids, 0)
    bea = jnp.where(valid, block_expert, block_expert[0])

    s2d = s.reshape(T // CHUNK, CHUNK)
    d2d = dest.reshape(T // CHUNK, CHUNK)

    x_pad = _shuffle_in(inp, s2d, d2d)
    y_pad = _grouped_mm(bxa, bea, x_pad, weight)
    out = _shuffle_out(y_pad, d2d, s2d)
    return out


# speedup vs baseline: 1.0118x; 1.0118x over previous
import functools

import jax
import jax.numpy as jnp
from jax import lax
from jax.experimental import pallas as pl
from jax.experimental.pallas import tpu as pltpu
from jax.experimental.pallas import tpu_sc as plsc

E = 64
DIN = 768
DOUT = 768
T = 32768
BT = 256
CAP = T + E * BT
G = CAP // BT

NW = 32
CHUNK = 64
NCH = T // NW // CHUNK


def _make_row_shuffle(n_table, n_out, d):
    mesh = plsc.VectorSubcoreMesh(core_axis_name="c", subcore_axis_name="s")

    @functools.partial(
        pl.kernel,
        mesh=mesh,
        out_type=jax.ShapeDtypeStruct((n_out, d), jnp.float32),
        scratch_types=[
            pltpu.VMEM((NCH, CHUNK), jnp.int32),
            pltpu.VMEM((NCH, CHUNK), jnp.int32),
            pltpu.VMEM((2, CHUNK, d), jnp.float32),
            pltpu.SemaphoreType.DMA,
            pltpu.SemaphoreType.DMA,
            pltpu.SemaphoreType.DMA,
        ],
    )
    def k(table_hbm, src_hbm, dst_hbm, out_hbm, src_v, dst_v, rows_v,
          isem, gsem, ssem):
        wid = lax.axis_index("s") * 2 + lax.axis_index("c")
        row0 = wid * NCH
        pltpu.async_copy(src_hbm.at[pl.ds(row0, NCH)], src_v, isem)
        pltpu.async_copy(dst_hbm.at[pl.ds(row0, NCH)], dst_v, isem).wait()
        pltpu.make_async_copy(src_hbm.at[pl.ds(row0, NCH)], src_v, isem).wait()

        pltpu.async_copy(table_hbm.at[src_v.at[0]], rows_v.at[0], gsem).wait()

        def body(c, _):
            buf = lax.rem(c, 2)
            nxt = lax.rem(c + 1, 2)

            @pl.when(c + 1 < NCH)
            def _():
                pltpu.async_copy(
                    table_hbm.at[src_v.at[c + 1]], rows_v.at[nxt], gsem
                )

            pltpu.async_copy(
                rows_v.at[buf], out_hbm.at[dst_v.at[c]], ssem
            ).wait()

            @pl.when(c + 1 < NCH)
            def _():
                pltpu.make_async_copy(
                    table_hbm.at[src_v.at[0]], rows_v.at[nxt], gsem
                ).wait()

            return ()

        lax.fori_loop(0, NCH, body, (), unroll=False)

    return k


_shuffle_in = _make_row_shuffle(T, CAP, DIN)
_shuffle_out = _make_row_shuffle(CAP, T, DOUT)


def _mm_body(bx_ref, be_ref, x_ref, w_ref, y_ref):
    y_ref[...] = lax.dot_general(
        x_ref[...], w_ref[0],
        dimension_numbers=(((1,), (1,)), ((), ())),
        preferred_element_type=jnp.float32,
    )


_grouped_mm = pl.pallas_call(
    _mm_body,
    grid_spec=pltpu.PrefetchScalarGridSpec(
        num_scalar_prefetch=2,
        grid=(G,),
        in_specs=[
            pl.BlockSpec((BT, DIN), lambda g, bx, be: (bx[g], 0)),
            pl.BlockSpec((1, DOUT, DIN), lambda g, bx, be: (be[g], 0, 0)),
        ],
        out_specs=pl.BlockSpec((BT, DOUT), lambda g, bx, be: (bx[g], 0)),
    ),
    out_shape=jax.ShapeDtypeStruct((CAP, DOUT), jnp.float32),
    compiler_params=pltpu.CompilerParams(
        dimension_semantics=("arbitrary",),
    ),
)


def kernel(inp, gate, weight):
    gate = gate.astype(jnp.int32)
    k = jnp.arange(T, dtype=jnp.int32)
    key = jnp.sort((gate << 16) | k)
    s = key & 0xFFFF
    sg = key >> 16
    counts = jnp.bincount(gate, length=E).astype(jnp.int32)
    raw_off = jnp.concatenate([jnp.zeros((1,), jnp.int32),
                               jnp.cumsum(counts)[:-1].astype(jnp.int32)])
    padded = ((counts + BT - 1) // BT) * BT
    pad_off = jnp.concatenate([jnp.zeros((1,), jnp.int32),
                               jnp.cumsum(padded)[:-1].astype(jnp.int32)])
    inc = padded - counts
    bnd = raw_off[1:]
    dest = k + jnp.sum(
        (bnd[None, :] <= k[:, None]) * inc[None, :-1],
        axis=1, dtype=jnp.int32)
    cum_blocks = jnp.cumsum(padded // BT).astype(jnp.int32)
    g_ids = jnp.arange(G, dtype=jnp.int32)
    block_expert = jnp.sum(
        cum_blocks[None, :] <= g_ids[:, None], axis=1, dtype=jnp.int32)
    block_expert = jnp.minimum(block_expert, E - 1)
    valid = g_ids < cum_blocks[-1]
    bxa = jnp.where(valid, g_ids, 0)
    bea = jnp.where(valid, block_expert, block_expert[0])

    s2d = s.reshape(T // CHUNK, CHUNK)
    d2d = dest.reshape(T // CHUNK, CHUNK)

    x_pad = _shuffle_in(inp, s2d, d2d)
    y_pad = _grouped_mm(bxa, bea, x_pad, weight)
    out = _shuffle_out(y_pad, d2d, s2d)
    return out
